# baseline (speedup 1.0000x reference)
import jax
import jax.numpy as jnp
from jax.experimental import pallas as pl
from jax.experimental.pallas import tpu as pltpu

T = 8192
D_IN = 4096
D_OUT = 4096
R = 16
E = 8
ER = E * R
KA = D_IN + ER
_SCALING = 32.0 / 16.0

BTA = 1024
BTB = 2048
BD = 512


def _gate_body(x_ref, ra_ref, logits_ref, xa_ref):
    x = x_ref[...]
    la = jax.lax.dot_general(
        x, ra_ref[...], (((1,), (1,)), ((), ())),
        preferred_element_type=jnp.float32)
    a = la[:, :ER]
    logits = la[:, ER:]
    logits_ref[...] = logits
    iota_e = jax.lax.broadcasted_iota(jnp.int32, (BTA, E), 1)
    v1 = jnp.max(logits, axis=1, keepdims=True)
    i1 = jnp.min(jnp.where(logits == v1, iota_e, E), axis=1, keepdims=True)
    masked = jnp.where(iota_e == i1, -jnp.inf, logits)
    v2 = jnp.max(masked, axis=1, keepdims=True)
    i2 = jnp.min(jnp.where(masked == v2, iota_e, E), axis=1, keepdims=True)
    ed = jnp.exp(v2 - v1)
    denom = 1.0 + ed
    g1 = 1.0 / denom
    g2 = ed / denom
    lane_e = jax.lax.broadcasted_iota(jnp.int32, (BTA, ER), 1) // R
    gate_x = (jnp.where(lane_e == i1, g1, 0.0)
              + jnp.where(lane_e == i2, g2, 0.0)) * _SCALING
    xa_ref[:, :D_IN] = x.astype(jnp.bfloat16)
    xa_ref[:, D_IN:] = (a * gate_x).astype(jnp.bfloat16)


def _main_body(xa_ref, wa_ref, b_ref, out_ref):
    out_ref[...] = jax.lax.dot_general(
        xa_ref[...], wa_ref[...], (((1,), (1,)), ((), ())),
        preferred_element_type=jnp.float32) + b_ref[...]


def kernel(x, base_W, base_b, router_W, lora_A, lora_B):
    a_all = lora_A.reshape(ER, D_IN)
    ra = jnp.concatenate([a_all, router_W], axis=0)
    ball_t = jnp.transpose(lora_B, (1, 0, 2)).reshape(D_OUT, ER)
    wa = jnp.concatenate([base_W, ball_t], axis=1).astype(jnp.bfloat16)
    bias = base_b.reshape(1, D_OUT)

    logits, xa = pl.pallas_call(
        _gate_body,
        grid=(T // BTA,),
        in_specs=[
            pl.BlockSpec((BTA, D_IN), lambda i: (i, 0)),
            pl.BlockSpec((ER + E, D_IN), lambda i: (0, 0)),
        ],
        out_specs=[
            pl.BlockSpec((BTA, E), lambda i: (i, 0)),
            pl.BlockSpec((BTA, KA), lambda i: (i, 0)),
        ],
        out_shape=[
            jax.ShapeDtypeStruct((T, E), jnp.float32),
            jax.ShapeDtypeStruct((T, KA), jnp.bfloat16),
        ],
        compiler_params=pltpu.CompilerParams(
            dimension_semantics=("parallel",)),
    )(x, ra)

    out = pl.pallas_call(
        _main_body,
        grid=(T // BTB, D_OUT // BD),
        in_specs=[
            pl.BlockSpec((BTB, KA), lambda i, j: (i, 0)),
            pl.BlockSpec((BD, KA), lambda i, j: (j, 0)),
            pl.BlockSpec((1, BD), lambda i, j: (0, j)),
        ],
        out_specs=pl.BlockSpec((BTB, BD), lambda i, j: (i, j)),
        out_shape=jax.ShapeDtypeStruct((T, D_OUT), jnp.float32),
        compiler_params=pltpu.CompilerParams(
            dimension_semantics=("parallel", "arbitrary")),
    )(xa, wa, bias)
    return out, logits


# speedup vs baseline: 1.0678x; 1.0678x over previous
import jax
import jax.numpy as jnp
from jax.experimental import pallas as pl
from jax.experimental.pallas import tpu as pltpu

T = 8192
D_IN = 4096
D_OUT = 4096
R = 16
E = 8
ER = E * R
KA = D_IN + ER
_SCALING = 32.0 / 16.0

BTA = 1024
BTB = 2048
BD = 256


def _gate_body(x_ref, ra_ref, logits_ref, xa_ref):
    x = x_ref[...]
    la = jax.lax.dot_general(
        x, ra_ref[...], (((1,), (1,)), ((), ())),
        preferred_element_type=jnp.float32)
    a = la[:, :ER]
    logits = la[:, ER:]
    logits_ref[...] = logits
    iota_e = jax.lax.broadcasted_iota(jnp.int32, (BTA, E), 1)
    v1 = jnp.max(logits, axis=1, keepdims=True)
    i1 = jnp.min(jnp.where(logits == v1, iota_e, E), axis=1, keepdims=True)
    masked = jnp.where(iota_e == i1, -jnp.inf, logits)
    v2 = jnp.max(masked, axis=1, keepdims=True)
    i2 = jnp.min(jnp.where(masked == v2, iota_e, E), axis=1, keepdims=True)
    ed = jnp.exp(v2 - v1)
    denom = 1.0 + ed
    g1 = 1.0 / denom
    g2 = ed / denom
    lane_e = jax.lax.broadcasted_iota(jnp.int32, (BTA, ER), 1) // R
    gate_x = (jnp.where(lane_e == i1, g1, 0.0)
              + jnp.where(lane_e == i2, g2, 0.0)) * _SCALING
    xa_ref[:, :D_IN] = x.astype(jnp.bfloat16)
    xa_ref[:, D_IN:] = (a * gate_x).astype(jnp.bfloat16)


def _main_body(xa_ref, w_ref, bt_ref, b_ref, out_ref):
    wa = jnp.concatenate(
        [w_ref[...].astype(jnp.bfloat16), bt_ref[...].astype(jnp.bfloat16)],
        axis=1)
    out_ref[...] = jax.lax.dot_general(
        xa_ref[...], wa, (((1,), (1,)), ((), ())),
        preferred_element_type=jnp.float32) + b_ref[...]


def kernel(x, base_W, base_b, router_W, lora_A, lora_B):
    a_all = lora_A.reshape(ER, D_IN)
    ra = jnp.concatenate([a_all, router_W], axis=0)
    ball_t = jnp.transpose(lora_B, (1, 0, 2)).reshape(D_OUT, ER)
    bias = base_b.reshape(1, D_OUT)

    logits, xa = pl.pallas_call(
        _gate_body,
        grid=(T // BTA,),
        in_specs=[
            pl.BlockSpec((BTA, D_IN), lambda i: (i, 0)),
            pl.BlockSpec((ER + E, D_IN), lambda i: (0, 0)),
        ],
        out_specs=[
            pl.BlockSpec((BTA, E), lambda i: (i, 0)),
            pl.BlockSpec((BTA, KA), lambda i: (i, 0)),
        ],
        out_shape=[
            jax.ShapeDtypeStruct((T, E), jnp.float32),
            jax.ShapeDtypeStruct((T, KA), jnp.bfloat16),
        ],
        compiler_params=pltpu.CompilerParams(
            dimension_semantics=("parallel",)),
    )(x, ra)

    out = pl.pallas_call(
        _main_body,
        grid=(T // BTB, D_OUT // BD),
        in_specs=[
            pl.BlockSpec((BTB, KA), lambda i, j: (i, 0)),
            pl.BlockSpec((BD, D_IN), lambda i, j: (j, 0)),
            pl.BlockSpec((BD, ER), lambda i, j: (j, 0)),
            pl.BlockSpec((1, BD), lambda i, j: (0, j)),
        ],
        out_specs=pl.BlockSpec((BTB, BD), lambda i, j: (i, j)),
        out_shape=jax.ShapeDtypeStruct((T, D_OUT), jnp.float32),
        compiler_params=pltpu.CompilerParams(
            dimension_semantics=("parallel", "arbitrary")),
    )(xa, base_W, ball_t, bias)
    return out, logits


# speedup vs baseline: 1.1492x; 1.0762x over previous
import jax
import jax.numpy as jnp
from jax.experimental import pallas as pl
from jax.experimental.pallas import tpu as pltpu

T = 8192
D_IN = 4096
D_OUT = 4096
R = 16
E = 8
ER = E * R
_SCALING = 32.0 / 16.0

BT = 1024
BD = 512


def _body(x_ref, w_ref, ra_ref, ball_ref, b_ref, out_ref, logits_ref, aw_ref):
    j = pl.program_id(1)

    @pl.when(j == 0)
    def _gating():
        x = x_ref[...]
        la = jax.lax.dot_general(
            x, ra_ref[...], (((1,), (1,)), ((), ())),
            preferred_element_type=jnp.float32)
        a = la[:, :ER]
        logits = la[:, ER:]
        logits_ref[...] = logits
        iota_e = jax.lax.broadcasted_iota(jnp.int32, (BT, E), 1)
        v1 = jnp.max(logits, axis=1, keepdims=True)
        i1 = jnp.min(jnp.where(logits == v1, iota_e, E), axis=1, keepdims=True)
        masked = jnp.where(iota_e == i1, -jnp.inf, logits)
        v2 = jnp.max(masked, axis=1, keepdims=True)
        i2 = jnp.min(jnp.where(masked == v2, iota_e, E), axis=1, keepdims=True)
        ed = jnp.exp(v2 - v1)
        denom = 1.0 + ed
        g1 = 1.0 / denom
        g2 = ed / denom
        lane_e = jax.lax.broadcasted_iota(jnp.int32, (BT, ER), 1) // R
        gate_x = (jnp.where(lane_e == i1, g1, 0.0)
                  + jnp.where(lane_e == i2, g2, 0.0)) * _SCALING
        aw_ref[...] = a * gate_x

    acc = jax.lax.dot_general(
        x_ref[...], w_ref[...], (((1,), (1,)), ((), ())),
        preferred_element_type=jnp.float32)
    acc += jnp.dot(aw_ref[...], ball_ref[...],
                   preferred_element_type=jnp.float32)
    out_ref[...] = acc + b_ref[...]


def kernel(x, base_W, base_b, router_W, lora_A, lora_B):
    a_all = lora_A.reshape(ER, D_IN)
    ra = jnp.concatenate([a_all, router_W], axis=0)
    b_all = jnp.transpose(lora_B, (0, 2, 1)).reshape(ER, D_OUT)
    bias = base_b.reshape(1, D_OUT)

    grid = (T // BT, D_OUT // BD)
    out, logits = pl.pallas_call(
        _body,
        grid=grid,
        in_specs=[
            pl.BlockSpec((BT, D_IN), lambda i, j: (i, 0)),
            pl.BlockSpec((BD, D_IN), lambda i, j: (j, 0)),
            pl.BlockSpec((ER + E, D_IN), lambda i, j: (0, 0)),
            pl.BlockSpec((ER, BD), lambda i, j: (0, j)),
            pl.BlockSpec((1, BD), lambda i, j: (0, j)),
        ],
        out_specs=[
            pl.BlockSpec((BT, BD), lambda i, j: (i, j)),
            pl.BlockSpec((BT, E), lambda i, j: (i, 0)),
        ],
        out_shape=[
            jax.ShapeDtypeStruct((T, D_OUT), jnp.float32),
            jax.ShapeDtypeStruct((T, E), jnp.float32),
        ],
        scratch_shapes=[pltpu.VMEM((BT, ER), jnp.float32)],
        compiler_params=pltpu.CompilerParams(
            dimension_semantics=("parallel", "arbitrary"),
            vmem_limit_bytes=128 * 1024 * 1024),
    )(x, base_W, ra, b_all, bias)
    return out, logits


# speedup vs baseline: 1.1504x; 1.0011x over previous
import jax
import jax.numpy as jnp
from jax.experimental import pallas as pl
from jax.experimental.pallas import tpu as pltpu

T = 8192
D_IN = 4096
D_OUT = 4096
R = 16
E = 8
ER = E * R
_SCALING = 32.0 / 16.0

BT = 1024
BD = 512


def _body(x_ref, w_ref, ra_ref, ball_ref, b_ref, out_ref, logits_ref, aw_ref):
    j = pl.program_id(1)

    @pl.when(j == 0)
    def _gating():
        x = x_ref[...]
        la = jax.lax.dot_general(
            x, ra_ref[...], (((1,), (1,)), ((), ())),
            preferred_element_type=jnp.float32)
        a = la[:, :ER]
        logits = la[:, ER:]
        logits_ref[...] = logits
        iota_e = jax.lax.broadcasted_iota(jnp.int32, (BT, E), 1)
        v1 = jnp.max(logits, axis=1, keepdims=True)
        i1 = jnp.min(jnp.where(logits == v1, iota_e, E), axis=1, keepdims=True)
        masked = jnp.where(iota_e == i1, -jnp.inf, logits)
        v2 = jnp.max(masked, axis=1, keepdims=True)
        i2 = jnp.min(jnp.where(masked == v2, iota_e, E), axis=1, keepdims=True)
        ed = jnp.exp(v2 - v1)
        denom = 1.0 + ed
        g1 = 1.0 / denom
        g2 = ed / denom
        lane_e = jax.lax.broadcasted_iota(jnp.int32, (BT, ER), 1) // R
        gate_x = (jnp.where(lane_e == i1, g1, 0.0)
                  + jnp.where(lane_e == i2, g2, 0.0)) * _SCALING
        aw_ref[...] = a * gate_x

    acc = jax.lax.dot_general(
        x_ref[...], w_ref[...], (((1,), (1,)), ((), ())),
        preferred_element_type=jnp.float32)
    acc += jnp.dot(aw_ref[...], ball_ref[...],
                   preferred_element_type=jnp.float32)
    out_ref[...] = acc + b_ref[...]


def kernel(x, base_W, base_b, router_W, lora_A, lora_B):
    a_all = lora_A.reshape(ER, D_IN)
    ra = jnp.concatenate([a_all, router_W], axis=0)
    b_all = jnp.transpose(lora_B, (0, 2, 1)).reshape(ER, D_OUT)
    bias = base_b.reshape(1, D_OUT)

    grid = (T // BT, D_OUT // BD)
    out, logits = pl.pallas_call(
        _body,
        grid=grid,
        in_specs=[
            pl.BlockSpec((BT, D_IN), lambda i, j: (i, 0)),
            pl.BlockSpec((BD, D_IN), lambda i, j: (j, 0)),
            pl.BlockSpec((ER + E, D_IN), lambda i, j: (0, 0)),
            pl.BlockSpec((ER, BD), lambda i, j: (0, j)),
            pl.BlockSpec((1, BD), lambda i, j: (0, j)),
        ],
        out_specs=[
            pl.BlockSpec((BT, BD), lambda i, j: (i, j)),
            pl.BlockSpec((BT, E), lambda i, j: (i, 0)),
        ],
        out_shape=[
            jax.ShapeDtypeStruct((T, D_OUT), jnp.float32),
            jax.ShapeDtypeStruct((T, E), jnp.float32),
        ],
        scratch_shapes=[pltpu.VMEM((BT, ER), jnp.float32)],
        compiler_params=pltpu.CompilerParams(
            dimension_semantics=("arbitrary", "arbitrary")),
    )(x, base_W, ra, b_all, bias)
    return out, logits
